# baseline (speedup 1.0000x reference)
import functools

import jax
import jax.numpy as jnp
from jax.experimental import pallas as pl
from jax.experimental.pallas import tpu as pltpu


def _proj_kernel(x_ref, w_ref, b_ref, o_ref):
    x_bf = x_ref[...].astype(jnp.bfloat16)
    acc = jax.lax.dot_general(
        x_bf, w_ref[...],
        dimension_numbers=(((1,), (0,)), ((), ())),
        preferred_element_type=jnp.float32,
    )
    o_ref[...] = acc + b_ref[...]


@functools.partial(jax.jit, static_argnames=("bm", "bn"))
def _proj(x, wt, b2, bm, bn):
    tot, enc = x.shape
    dec = wt.shape[1]
    return pl.pallas_call(
        _proj_kernel,
        grid=(tot // bm, dec // bn),
        in_specs=[
            pl.BlockSpec((bm, enc), lambda i, j: (i, 0)),
            pl.BlockSpec((enc, bn), lambda i, j: (0, j)),
            pl.BlockSpec((1, bn), lambda i, j: (0, j)),
        ],
        out_specs=pl.BlockSpec((bm, bn), lambda i, j: (i, j)),
        out_shape=jax.ShapeDtypeStruct((tot, dec), jnp.float32),
        compiler_params=pltpu.CompilerParams(
            dimension_semantics=("arbitrary", "arbitrary"),
            vmem_limit_bytes=100 * 1024 * 1024,
        ),
    )(x, wt, b2)


def kernel(x, W, b):
    wt = W.astype(jnp.bfloat16).T
    b2 = b[None, :]
    return _proj(x, wt, b2, bm=2048, bn=896)


# speedup vs baseline: 1.2944x; 1.2944x over previous
import functools

import jax
import jax.numpy as jnp
from jax.experimental import pallas as pl
from jax.experimental.pallas import tpu as pltpu


def _proj_kernel(x_ref, w_ref, b_ref, o_ref):
    x_bf = x_ref[...].astype(jnp.bfloat16)
    acc = jax.lax.dot_general(
        x_bf, w_ref[...],
        dimension_numbers=(((1,), (0,)), ((), ())),
        preferred_element_type=jnp.float32,
    )
    o_ref[...] = acc + b_ref[...]


@functools.partial(jax.jit, static_argnames=("bm",))
def _proj(x, wt, b2, bm):
    tot, enc = x.shape
    dec = wt.shape[1]
    return pl.pallas_call(
        _proj_kernel,
        grid=(tot // bm,),
        in_specs=[
            pl.BlockSpec((bm, enc), lambda i: (i, 0)),
            pl.BlockSpec((enc, dec), lambda i: (0, 0)),
            pl.BlockSpec((1, dec), lambda i: (0, 0)),
        ],
        out_specs=pl.BlockSpec((bm, dec), lambda i: (i, 0)),
        out_shape=jax.ShapeDtypeStruct((tot, dec), jnp.float32),
        compiler_params=pltpu.CompilerParams(
            dimension_semantics=("arbitrary",),
            vmem_limit_bytes=100 * 1024 * 1024,
        ),
    )(x, wt, b2)


def kernel(x, W, b):
    wt = W.astype(jnp.bfloat16).T
    b2 = b[None, :]
    return _proj(x, wt, b2, bm=512)


# speedup vs baseline: 1.3199x; 1.0196x over previous
import functools

import jax
import jax.numpy as jnp
from jax.experimental import pallas as pl
from jax.experimental.pallas import tpu as pltpu


def _proj_kernel(x_ref, w_ref, b_ref, o_ref):
    x_bf = x_ref[...].astype(jnp.bfloat16)
    acc = jax.lax.dot_general(
        x_bf, w_ref[...],
        dimension_numbers=(((1,), (0,)), ((), ())),
        preferred_element_type=jnp.float32,
    )
    o_ref[...] = acc + b_ref[...]


@functools.partial(jax.jit, static_argnames=("bm",))
def _proj(x, wt, b2, bm):
    tot, enc = x.shape
    dec = wt.shape[1]
    return pl.pallas_call(
        _proj_kernel,
        grid=(tot // bm,),
        in_specs=[
            pl.BlockSpec((bm, enc), lambda i: (i, 0)),
            pl.BlockSpec((enc, dec), lambda i: (0, 0)),
            pl.BlockSpec((1, dec), lambda i: (0, 0)),
        ],
        out_specs=pl.BlockSpec((bm, dec), lambda i: (i, 0)),
        out_shape=jax.ShapeDtypeStruct((tot, dec), jnp.float32),
        compiler_params=pltpu.CompilerParams(
            dimension_semantics=("arbitrary",),
            vmem_limit_bytes=100 * 1024 * 1024,
        ),
    )(x, wt, b2)


def kernel(x, W, b):
    wt = W.astype(jnp.bfloat16).T
    b2 = b[None, :]
    return _proj(x, wt, b2, bm=1024)
